# baseline (speedup 1.0000x reference)
import functools

import jax
import jax.numpy as jnp
from jax import lax
from jax.experimental import pallas as pl
from jax.experimental.pallas import tpu as pltpu
from jax.experimental.pallas import tpu_sc as plsc

M = 100000
D = 128
B = 16384

NC = 2
NS = 16
NW = NC * NS
PER_W = B // NW
CHUNK = 128
NCHUNK = PER_W // CHUNK

GCH = 64
WBLK = 2048
NWBLK = B // WBLK

_mesh = plsc.VectorSubcoreMesh(
    core_axis_name="c", subcore_axis_name="s", num_cores=NC, num_subcores=NS)
_sc_params = pltpu.CompilerParams(needs_layout_passes=False)


def _wid():
  return lax.axis_index("s") * NC + lax.axis_index("c")


CPW = 3224
CPREM = M - 31 * CPW


@functools.partial(
    pl.kernel,
    out_type=(
        jax.ShapeDtypeStruct((B, D), jnp.float32),
        jax.ShapeDtypeStruct((B,), jnp.int32),
        jax.ShapeDtypeStruct((M, D), jnp.float32),
    ),
    mesh=_mesh,
    compiler_params=_sc_params,
    scratch_types=[
        pltpu.VMEM((M,), jnp.int32),
        pltpu.VMEM((WBLK,), jnp.int32),
        pltpu.VMEM((WBLK,), jnp.int32),
        pltpu.VMEM((WBLK,), jnp.int32),
        pltpu.VMEM((WBLK,), jnp.int32),
        pltpu.VMEM((2 * PER_W,), jnp.int32),
        pltpu.VMEM((2, GCH, D), jnp.float32),
        pltpu.SemaphoreType.DMA,
        pltpu.SemaphoreType.DMA,
        pltpu.SemaphoreType.DMA,
        pltpu.SemaphoreType.DMA,
    ],
)
def _sc_gather_winners(mem_hbm, idx_hbm, h_hbm, w_hbm, out_hbm,
                       aux, iblk0, iblk1, wblk0, wblk1, idx_v, rows_v,
                       sem_i, sem_g, sem_o, sem_c):
  w = _wid()
  iota16 = lax.iota(jnp.int32, 16)
  iblks = (iblk0, iblk1)
  wblks = (wblk0, wblk1)

  @pl.when(w == 0)
  def _winners():
    rem_cp = pltpu.async_copy(
        mem_hbm.at[pl.ds(31 * CPW, CPREM)],
        out_hbm.at[pl.ds(31 * CPW, CPREM)], sem_c)
    cp0 = pltpu.async_copy(idx_hbm.at[pl.ds(0, WBLK)], iblks[0], sem_i)
    pending = cp0
    for b in range(NWBLK):
      nxt = None
      if b + 1 < NWBLK:
        nxt = pltpu.async_copy(
            idx_hbm.at[pl.ds((b + 1) * WBLK, WBLK)], iblks[(b + 1) % 2], sem_i)
      pending.wait()
      pending = nxt
      base = b * WBLK
      blk = iblks[b % 2]

      def phase_a(k, carry, blk=blk, base=base):
        for u in range(4):
          off = k * 64 + u * 16
          idx_c = blk[pl.ds(off, 16)]
          _counts, last = plsc.scan_count(idx_c)
          plsc.store_scatter(aux, [idx_c], base + off + iota16, mask=last)
        return carry

      lax.fori_loop(0, WBLK // 64, phase_a, None)

    cp0 = pltpu.async_copy(idx_hbm.at[pl.ds(0, WBLK)], iblks[0], sem_i)
    pending = cp0
    out_cp = [None, None]
    for b in range(NWBLK):
      nxt = None
      if b + 1 < NWBLK:
        nxt = pltpu.async_copy(
            idx_hbm.at[pl.ds((b + 1) * WBLK, WBLK)], iblks[(b + 1) % 2], sem_i)
      pending.wait()
      pending = nxt
      if out_cp[b % 2] is not None:
        out_cp[b % 2].wait()
      blk = iblks[b % 2]
      wb = wblks[b % 2]

      def phase_b(k, carry, blk=blk, wb=wb):
        for u in range(4):
          off = k * 64 + u * 16
          idx_c = blk[pl.ds(off, 16)]
          wb[pl.ds(off, 16)] = plsc.load_gather(aux, [idx_c])
        return carry

      lax.fori_loop(0, WBLK // 64, phase_b, None)
      out_cp[b % 2] = pltpu.async_copy(
          wb, w_hbm.at[pl.ds(b * WBLK, WBLK)], sem_o)
    for cp in out_cp:
      if cp is not None:
        cp.wait()
    rem_cp.wait()

  @pl.when(w > 0)
  def _gather():
    cstart = pl.multiple_of((w - 1) * CPW, 8)
    mem_cp = pltpu.async_copy(
        mem_hbm.at[pl.ds(cstart, CPW)], out_hbm.at[pl.ds(cstart, CPW)], sem_c)
    start = pl.multiple_of((w - 1) * PER_W, PER_W)

    def run(start, n, voff):
      pltpu.sync_copy(
          idx_hbm.at[pl.ds(start, n * GCH)],
          idx_v.at[pl.ds(voff * GCH, n * GCH)])
      cps = [None, None]
      for j in range(n):
        if cps[j % 2] is not None:
          cps[j % 2].wait()
        g = pltpu.async_copy(
            mem_hbm.at[idx_v.at[pl.ds((voff + j) * GCH, GCH)]],
            rows_v.at[j % 2], sem_g)
        g.wait()
        cps[j % 2] = pltpu.async_copy(
            rows_v.at[j % 2], h_hbm.at[pl.ds(start + j * GCH, GCH)], sem_o)
      for cp in cps:
        if cp is not None:
          cp.wait()

    run(start, PER_W // GCH, 0)

    @pl.when(w == NW - 1)
    def _extra():
      run((NW - 1) * PER_W, PER_W // GCH, 8)

    mem_cp.wait()


_BLK = 2048


def _gru_body(val_ref, h_ref, wt_ref, ut_ref, bih_ref, bhh_ref, out_ref):
  v = val_ref[...]
  h = h_ref[...]
  gi = jnp.dot(v, wt_ref[...], preferred_element_type=jnp.float32) + bih_ref[...]
  gh = jnp.dot(h, ut_ref[...], preferred_element_type=jnp.float32) + bhh_ref[...]
  i_r = gi[:, :D]
  i_z = gi[:, D:2 * D]
  i_n = gi[:, 2 * D:]
  h_r = gh[:, :D]
  h_z = gh[:, D:2 * D]
  h_n = gh[:, 2 * D:]
  r = jax.nn.sigmoid(i_r + h_r)
  z = jax.nn.sigmoid(i_z + h_z)
  n = jnp.tanh(i_n + r * h_n)
  out_ref[...] = (1.0 - z) * n + z * h


def _tc_gru(val, h, wt, ut, bih, bhh):
  return pl.pallas_call(
      _gru_body,
      grid=(B // _BLK,),
      in_specs=[
          pl.BlockSpec((_BLK, D), lambda i: (i, 0)),
          pl.BlockSpec((_BLK, D), lambda i: (i, 0)),
          pl.BlockSpec((D, 3 * D), lambda i: (0, 0)),
          pl.BlockSpec((D, 3 * D), lambda i: (0, 0)),
          pl.BlockSpec((1, 3 * D), lambda i: (0, 0)),
          pl.BlockSpec((1, 3 * D), lambda i: (0, 0)),
      ],
      out_specs=pl.BlockSpec((_BLK, D), lambda i: (i, 0)),
      out_shape=jax.ShapeDtypeStruct((B, D), jnp.float32),
  )(val, h, wt, ut, bih, bhh)


@functools.partial(
    pl.kernel,
    out_type=(),
    mesh=_mesh,
    compiler_params=_sc_params,
    scratch_types=[
        pltpu.VMEM((NCHUNK, CHUNK), jnp.int32),
        pltpu.VMEM((NCHUNK, CHUNK), jnp.int32),
        pltpu.VMEM((PER_W, D), jnp.float32),
        pltpu.SemaphoreType.DMA,
    ],
)
def _sc_scatter(hnew_hbm, idx_hbm, win_hbm, out_hbm, idx_v, win_v, rows_v, sem):
  w = _wid()
  base = pl.multiple_of(w * PER_W, PER_W)
  for r in range(NCHUNK):
    pltpu.sync_copy(idx_hbm.at[pl.ds(base + r * CHUNK, CHUNK)], idx_v.at[r])
    pltpu.sync_copy(win_hbm.at[pl.ds(base + r * CHUNK, CHUNK)], win_v.at[r])
  copies = []
  for j in range(NCHUNK):
    copies.append(pltpu.async_copy(
        hnew_hbm.at[win_v.at[j]], rows_v.at[pl.ds(j * CHUNK, CHUNK)], sem))
  for c in copies:
    c.wait()
  copies = []
  for j in range(NCHUNK):
    copies.append(pltpu.async_copy(
        rows_v.at[pl.ds(j * CHUNK, CHUNK)], out_hbm.at[idx_v.at[j]], sem))
  for c in copies:
    c.wait()


def kernel(mem, idx, val, W_ih, W_hh, b_ih, b_hh):
  idx = idx.astype(jnp.int32)

  h, win, outbuf = _sc_gather_winners(mem, idx)

  out_ref = jax.new_ref(mem)

  wt = W_ih.T
  ut = W_hh.T
  bih = b_ih.reshape(1, 3 * D)
  bhh = b_hh.reshape(1, 3 * D)
  h_new = _tc_gru(val, h, wt, ut, bih, bhh)

  _sc_scatter(h_new, idx, win, out_ref)
  return out_ref[...]


# speedup vs baseline: 17.0919x; 17.0919x over previous
import functools

import jax
import jax.numpy as jnp
from jax import lax
from jax.experimental import pallas as pl
from jax.experimental.pallas import tpu as pltpu
from jax.experimental.pallas import tpu_sc as plsc

M = 100000
D = 128
B = 16384

NC = 2
NS = 16
NW = NC * NS
PER_W = B // NW
CHUNK = 128
NCHUNK = PER_W // CHUNK

GCH = 64
WBLK = 2048
NWBLK = B // WBLK

_mesh = plsc.VectorSubcoreMesh(
    core_axis_name="c", subcore_axis_name="s", num_cores=NC, num_subcores=NS)
_sc_params = pltpu.CompilerParams(needs_layout_passes=False)


def _wid():
  return lax.axis_index("s") * NC + lax.axis_index("c")


@functools.partial(
    pl.kernel,
    out_type=(
        jax.ShapeDtypeStruct((B, D), jnp.float32),
        jax.ShapeDtypeStruct((B,), jnp.int32),
    ),
    mesh=_mesh,
    compiler_params=_sc_params,
    scratch_types=[
        pltpu.VMEM((M,), jnp.int32),
        pltpu.VMEM((WBLK,), jnp.int32),
        pltpu.VMEM((WBLK,), jnp.int32),
        pltpu.VMEM((WBLK,), jnp.int32),
        pltpu.VMEM((WBLK,), jnp.int32),
        pltpu.VMEM((2 * PER_W,), jnp.int32),
        pltpu.VMEM((2, GCH, D), jnp.float32),
        pltpu.SemaphoreType.DMA,
        pltpu.SemaphoreType.DMA,
        pltpu.SemaphoreType.DMA,
    ],
)
def _sc_gather_winners(mem_hbm, idx_hbm, h_hbm, w_hbm,
                       aux, iblk0, iblk1, wblk0, wblk1, idx_v, rows_v,
                       sem_i, sem_g, sem_o):
  w = _wid()
  iota16 = lax.iota(jnp.int32, 16)
  iblks = (iblk0, iblk1)
  wblks = (wblk0, wblk1)

  @pl.when(w == 0)
  def _winners():
    pending = pltpu.async_copy(idx_hbm.at[pl.ds(0, WBLK)], iblks[0], sem_i)
    for b in range(NWBLK):
      nxt = None
      if b + 1 < NWBLK:
        nxt = pltpu.async_copy(
            idx_hbm.at[pl.ds((b + 1) * WBLK, WBLK)], iblks[(b + 1) % 2], sem_i)
      pending.wait()
      pending = nxt
      base = b * WBLK
      blk = iblks[b % 2]

      def phase_a(k, carry, blk=blk, base=base):
        for u in range(4):
          off = k * 64 + u * 16
          idx_c = blk[pl.ds(off, 16)]
          _counts, last = plsc.scan_count(idx_c)
          plsc.store_scatter(aux, [idx_c], base + off + iota16, mask=last)
        return carry

      lax.fori_loop(0, WBLK // 64, phase_a, None)

    pending = pltpu.async_copy(idx_hbm.at[pl.ds(0, WBLK)], iblks[0], sem_i)
    out_cp = [None, None]
    for b in range(NWBLK):
      nxt = None
      if b + 1 < NWBLK:
        nxt = pltpu.async_copy(
            idx_hbm.at[pl.ds((b + 1) * WBLK, WBLK)], iblks[(b + 1) % 2], sem_i)
      pending.wait()
      pending = nxt
      if out_cp[b % 2] is not None:
        out_cp[b % 2].wait()
      blk = iblks[b % 2]
      wb = wblks[b % 2]

      def phase_b(k, carry, blk=blk, wb=wb):
        for u in range(4):
          off = k * 64 + u * 16
          idx_c = blk[pl.ds(off, 16)]
          wb[pl.ds(off, 16)] = plsc.load_gather(aux, [idx_c])
        return carry

      lax.fori_loop(0, WBLK // 64, phase_b, None)
      out_cp[b % 2] = pltpu.async_copy(
          wb, w_hbm.at[pl.ds(b * WBLK, WBLK)], sem_o)
    for cp in out_cp:
      if cp is not None:
        cp.wait()

  @pl.when(w > 0)
  def _gather():
    start = pl.multiple_of((w - 1) * PER_W, PER_W)

    def run(start, n, voff):
      pltpu.sync_copy(
          idx_hbm.at[pl.ds(start, n * GCH)],
          idx_v.at[pl.ds(voff * GCH, n * GCH)])
      cps = [None, None]
      for j in range(n):
        if cps[j % 2] is not None:
          cps[j % 2].wait()
        g = pltpu.async_copy(
            mem_hbm.at[idx_v.at[pl.ds((voff + j) * GCH, GCH)]],
            rows_v.at[j % 2], sem_g)
        g.wait()
        cps[j % 2] = pltpu.async_copy(
            rows_v.at[j % 2], h_hbm.at[pl.ds(start + j * GCH, GCH)], sem_o)
      for cp in cps:
        if cp is not None:
          cp.wait()

    run(start, PER_W // GCH, 0)

    @pl.when(w == NW - 1)
    def _extra():
      run((NW - 1) * PER_W, PER_W // GCH, 8)


_BLK = 2048


def _gru_body(val_ref, h_ref, wt_ref, ut_ref, bih_ref, bhh_ref, out_ref):
  v = val_ref[...]
  h = h_ref[...]
  gi = jnp.dot(v, wt_ref[...], preferred_element_type=jnp.float32) + bih_ref[...]
  gh = jnp.dot(h, ut_ref[...], preferred_element_type=jnp.float32) + bhh_ref[...]
  i_r = gi[:, :D]
  i_z = gi[:, D:2 * D]
  i_n = gi[:, 2 * D:]
  h_r = gh[:, :D]
  h_z = gh[:, D:2 * D]
  h_n = gh[:, 2 * D:]
  r = jax.nn.sigmoid(i_r + h_r)
  z = jax.nn.sigmoid(i_z + h_z)
  n = jnp.tanh(i_n + r * h_n)
  out_ref[...] = (1.0 - z) * n + z * h


def _tc_gru(val, h, wt, ut, bih, bhh):
  return pl.pallas_call(
      _gru_body,
      grid=(B // _BLK,),
      in_specs=[
          pl.BlockSpec((_BLK, D), lambda i: (i, 0)),
          pl.BlockSpec((_BLK, D), lambda i: (i, 0)),
          pl.BlockSpec((D, 3 * D), lambda i: (0, 0)),
          pl.BlockSpec((D, 3 * D), lambda i: (0, 0)),
          pl.BlockSpec((1, 3 * D), lambda i: (0, 0)),
          pl.BlockSpec((1, 3 * D), lambda i: (0, 0)),
      ],
      out_specs=pl.BlockSpec((_BLK, D), lambda i: (i, 0)),
      out_shape=jax.ShapeDtypeStruct((B, D), jnp.float32),
  )(val, h, wt, ut, bih, bhh)


@functools.partial(
    pl.kernel,
    out_type=(),
    mesh=_mesh,
    compiler_params=_sc_params,
    scratch_types=[
        pltpu.VMEM((NCHUNK, CHUNK), jnp.int32),
        pltpu.VMEM((NCHUNK, CHUNK), jnp.int32),
        pltpu.VMEM((PER_W, D), jnp.float32),
        pltpu.SemaphoreType.DMA,
        pltpu.SemaphoreType.DMA,
    ],
)
def _sc_scatter(hnew_hbm, idx_hbm, win_hbm, out_hbm, idx_v, win_v, rows_v,
                sem_i, sem):
  w = _wid()
  base = pl.multiple_of(w * PER_W, PER_W)
  icps = []
  for r in range(NCHUNK):
    icps.append(pltpu.async_copy(
        idx_hbm.at[pl.ds(base + r * CHUNK, CHUNK)], idx_v.at[r], sem_i))
    icps.append(pltpu.async_copy(
        win_hbm.at[pl.ds(base + r * CHUNK, CHUNK)], win_v.at[r], sem_i))
  for cp in icps:
    cp.wait()
  copies = []
  for j in range(NCHUNK):
    copies.append(pltpu.async_copy(
        hnew_hbm.at[win_v.at[j]], rows_v.at[pl.ds(j * CHUNK, CHUNK)], sem))
  for c in copies:
    c.wait()
  copies = []
  for j in range(NCHUNK):
    copies.append(pltpu.async_copy(
        rows_v.at[pl.ds(j * CHUNK, CHUNK)], out_hbm.at[idx_v.at[j]], sem))
  for c in copies:
    c.wait()


def kernel(mem, idx, val, W_ih, W_hh, b_ih, b_hh):
  idx = idx.astype(jnp.int32)

  h, win = _sc_gather_winners(mem, idx)

  out_ref = jax.new_ref(mem)

  wt = W_ih.T
  ut = W_hh.T
  bih = b_ih.reshape(1, 3 * D)
  bhh = b_hh.reshape(1, 3 * D)
  h_new = _tc_gru(val, h, wt, ut, bih, bhh)

  _sc_scatter(h_new, idx, win, out_ref)
  return out_ref[...]
